# Initial kernel scaffold; baseline (speedup 1.0000x reference)
import jax
import jax.numpy as jnp
from jax.experimental import pallas as pl


def kernel(x, w1c, b1c, w2c, b2c, w1f, b1f, w2f, b2f):
    raise NotImplementedError("write your pallas kernel here")


# speedup vs baseline: 9.3319x; 9.3319x over previous
import jax
import jax.numpy as jnp
from jax.experimental import pallas as pl
from jax.experimental.pallas import tpu as pltpu


def _conv_kernel(xq_ref, w1_ref, b1_ref, w2_ref, b2_ref, o_ref, y1_scr):
    pid = pl.program_id(0)
    i0 = pid * 7

    def c1_body(c, carry):
        for ph in range(4):
            for pw in range(4):
                best = None
                for u in range(2):
                    for v in range(2):
                        acc = None
                        for di in range(3):
                            for dj in range(3):
                                sh = 4 * ph + 2 * u + di
                                sw = 4 * pw + 2 * v + dj
                                win = xq_ref[sh % 16, sw % 16,
                                             pl.ds(i0 + sh // 16, 8),
                                             pl.ds(sw // 16, 14), :]
                                t = w1_ref[c, 3 * di + dj] * win
                                acc = t if acc is None else acc + t
                        best = acc if best is None else jnp.maximum(best, acc)
                val = jnp.maximum(best + b1_ref[c, 0], 0.0)
                y1_scr[ph, pw, c, :, pl.ds(0, 14), :] = val
        return carry

    jax.lax.fori_loop(0, 16, c1_body, 0)

    def c2_body(c2, carry):
        accs = []
        for u2 in range(2):
            for v2 in range(2):
                acc = None
                for c in range(16):
                    for di in range(3):
                        for dj in range(3):
                            th = 2 * u2 + di
                            tw = 2 * v2 + dj
                            win = y1_scr[th % 4, tw % 4, c,
                                         pl.ds(th // 4, 7),
                                         pl.ds(tw // 4, 13), :]
                            t = w2_ref[c2, c * 9 + di * 3 + dj] * win
                            acc = t if acc is None else acc + t
                accs.append(acc)
        best = jnp.maximum(jnp.maximum(accs[0], accs[1]),
                           jnp.maximum(accs[2], accs[3]))
        val = jnp.maximum(best + b2_ref[c2, 0], 0.0)
        val = jnp.concatenate(
            [val, jnp.zeros((val.shape[0], 3, val.shape[2]), val.dtype)],
            axis=1)
        o_ref[c2, :, :, :] = val
        return carry

    jax.lax.fori_loop(0, 32, c2_body, 0)


def _head_kernel(y2_ref, w1_ref, b1_ref, w2_ref, b2_ref, o_ref):
    n = y2_ref.shape[-1]
    f = y2_ref[:, pl.ds(0, 13), :, :].reshape(32 * 13 * 16, n)
    z = jnp.dot(w1_ref[...], f, preferred_element_type=jnp.float32)
    z = jnp.maximum(z + b1_ref[...], 0.0)
    logits = jnp.dot(w2_ref[...], z,
                     preferred_element_type=jnp.float32) + b2_ref[...]
    m = jnp.max(logits, axis=0, keepdims=True)
    e = jnp.exp(logits - m)
    o_ref[...] = e / jnp.sum(e, axis=0, keepdims=True)


def kernel(x, w1c, b1c, w2c, b2c, w1f, b1f, w2f, b2f):
    N = x.shape[0]
    xr = x.reshape(N, 14, 16, 14, 16).transpose(2, 4, 1, 3, 0)
    xq = jnp.pad(xr, ((0, 0), (0, 0), (0, 2), (0, 2), (0, 0)))

    y2 = pl.pallas_call(
        _conv_kernel,
        out_shape=jax.ShapeDtypeStruct((32, 14, 16, N), jnp.float32),
        grid=(2,),
        in_specs=[
            pl.BlockSpec((16, 16, 16, 16, N), lambda h: (0, 0, 0, 0, 0)),
            pl.BlockSpec(memory_space=pltpu.MemorySpace.SMEM),
            pl.BlockSpec(memory_space=pltpu.MemorySpace.SMEM),
            pl.BlockSpec(memory_space=pltpu.MemorySpace.SMEM),
            pl.BlockSpec(memory_space=pltpu.MemorySpace.SMEM),
        ],
        out_specs=pl.BlockSpec((32, 7, 16, N), lambda h: (0, h, 0, 0)),
        scratch_shapes=[pltpu.VMEM((4, 4, 16, 8, 16, N), jnp.float32)],
        compiler_params=pltpu.CompilerParams(
            dimension_semantics=("parallel",),
            vmem_limit_bytes=110 * 1024 * 1024),
    )(xq, w1c, b1c, w2c, b2c)

    w1m = jnp.pad(w1f.reshape(10, 32, 13, 13),
                  ((0, 0), (0, 0), (0, 0), (0, 3))).reshape(10, 32 * 13 * 16)
    probs = pl.pallas_call(
        _head_kernel,
        out_shape=jax.ShapeDtypeStruct((2, N), jnp.float32),
        in_specs=[pl.BlockSpec(memory_space=pltpu.MemorySpace.VMEM)] * 5,
        out_specs=pl.BlockSpec(memory_space=pltpu.MemorySpace.VMEM),
        compiler_params=pltpu.CompilerParams(
            vmem_limit_bytes=32 * 1024 * 1024),
    )(y2, w1m, b1f.reshape(10, 1), w2f, b2f.reshape(2, 1))
    return probs.T


# speedup vs baseline: 18.4069x; 1.9725x over previous
import jax
import jax.numpy as jnp
from jax.experimental import pallas as pl
from jax.experimental.pallas import tpu as pltpu


def _conv_kernel(xq_ref, w1_ref, b1_ref, wa0_ref, wa1_ref, wb1_ref, b2_ref,
                 o_ref, y1_scr):
    n = xq_ref.shape[-1]

    def c1_body(c, carry):
        for ph in range(4):
            for pw in range(4):
                best = None
                for u in range(2):
                    for v in range(2):
                        acc = None
                        for di in range(3):
                            for dj in range(3):
                                sh = 4 * ph + 2 * u + di
                                sw = 4 * pw + 2 * v + dj
                                win = xq_ref[sh % 16, sw % 16,
                                             pl.ds(sh // 16, 8),
                                             pl.ds(sw // 16, 14), :]
                                t = w1_ref[c, 3 * di + dj] * win
                                acc = t if acc is None else acc + t
                        best = acc if best is None else jnp.maximum(best, acc)
                val = jnp.maximum(best + b1_ref[c, 0], 0.0)
                val = jnp.concatenate(
                    [val, jnp.zeros((8, 2, n), jnp.float32)],
                    axis=1).astype(jnp.bfloat16)
                y1_scr[ph, pw, c, :, :, :] = val
        return carry

    jax.lax.fori_loop(0, 16, c1_body, 0)

    m = 7 * 16 * n

    def rhs(u2, v2, di, dj):
        th = 2 * u2 + di
        tw = 2 * v2 + dj
        sl = y1_scr[th % 4, tw % 4, :, pl.ds(th // 4, 7), :, :]
        return sl.reshape(16, m)

    accs = []
    for u2 in range(2):
        for v2 in range(2):
            if v2 == 0:
                r = jnp.concatenate(
                    [rhs(u2, v2, di, dj)
                     for di in range(3) for dj in range(3)], axis=0)
                out = jnp.dot(wa0_ref[...], r,
                              preferred_element_type=jnp.float32)
                acc = out.reshape(32, 7, 16, n)[:, :, 0:13, :]
            else:
                ra = jnp.concatenate(
                    [rhs(u2, v2, di, dj)
                     for di in range(3) for dj in range(2)], axis=0)
                rb = jnp.concatenate(
                    [rhs(u2, v2, di, 2) for di in range(3)], axis=0)
                oa = jnp.dot(wa1_ref[...], ra,
                             preferred_element_type=jnp.float32)
                ob = jnp.dot(wb1_ref[...], rb,
                             preferred_element_type=jnp.float32)
                acc = (oa.reshape(32, 7, 16, n)[:, :, 0:13, :]
                       + ob.reshape(32, 7, 16, n)[:, :, 1:14, :])
            accs.append(acc)
    best = jnp.maximum(jnp.maximum(accs[0], accs[1]),
                       jnp.maximum(accs[2], accs[3]))
    val = jnp.maximum(best + b2_ref[...], 0.0)
    val = jnp.concatenate(
        [val.astype(jnp.bfloat16),
         jnp.zeros((32, 7, 3, n), jnp.bfloat16)], axis=2)
    o_ref[...] = val


def _head_kernel(y2_ref, w1_ref, b1_ref, w2_ref, b2_ref, o_ref):
    n = y2_ref.shape[-1]
    f = y2_ref[:, pl.ds(0, 13), :, :].reshape(32 * 13 * 16, n)
    z = jnp.dot(w1_ref[...], f, preferred_element_type=jnp.float32)
    z = jnp.maximum(z + b1_ref[...], 0.0)
    logits = jnp.dot(w2_ref[...], z,
                     preferred_element_type=jnp.float32) + b2_ref[...]
    mx = jnp.max(logits, axis=0, keepdims=True)
    e = jnp.exp(logits - mx)
    o_ref[...] = e / jnp.sum(e, axis=0, keepdims=True)


def kernel(x, w1c, b1c, w2c, b2c, w1f, b1f, w2f, b2f):
    N = x.shape[0]
    xr = x.reshape(N, 14, 16, 14, 16).transpose(2, 4, 1, 3, 0)
    xq = jnp.pad(xr, ((0, 0), (0, 0), (0, 2), (0, 2), (0, 0)))
    xqs = jnp.concatenate([xq[:, :, 0:9], xq[:, :, 7:16]], axis=2)

    w2r = w2c.reshape(32, 16, 3, 3).transpose(0, 2, 3, 1)
    wa0 = w2r.reshape(32, 144).astype(jnp.bfloat16)
    wa1 = w2r[:, :, 0:2, :].reshape(32, 96).astype(jnp.bfloat16)
    wb1 = w2r[:, :, 2, :].reshape(32, 48).astype(jnp.bfloat16)

    y2 = pl.pallas_call(
        _conv_kernel,
        out_shape=jax.ShapeDtypeStruct((32, 14, 16, N), jnp.bfloat16),
        grid=(2,),
        in_specs=[
            pl.BlockSpec((16, 16, 9, 16, N), lambda h: (0, 0, h, 0, 0)),
            pl.BlockSpec(memory_space=pltpu.MemorySpace.SMEM),
            pl.BlockSpec(memory_space=pltpu.MemorySpace.SMEM),
            pl.BlockSpec(memory_space=pltpu.MemorySpace.VMEM),
            pl.BlockSpec(memory_space=pltpu.MemorySpace.VMEM),
            pl.BlockSpec(memory_space=pltpu.MemorySpace.VMEM),
            pl.BlockSpec(memory_space=pltpu.MemorySpace.VMEM),
        ],
        out_specs=pl.BlockSpec((32, 7, 16, N), lambda h: (0, h, 0, 0)),
        scratch_shapes=[pltpu.VMEM((4, 4, 16, 8, 16, N), jnp.bfloat16)],
        compiler_params=pltpu.CompilerParams(
            dimension_semantics=("parallel",),
            vmem_limit_bytes=64 * 1024 * 1024),
    )(xqs, w1c, b1c, wa0, wa1, wb1, b2c.reshape(32, 1, 1, 1))

    w1m = jnp.pad(w1f.reshape(10, 32, 13, 13),
                  ((0, 0), (0, 0), (0, 0), (0, 3))
                  ).reshape(10, 32 * 13 * 16).astype(jnp.bfloat16)
    probs = pl.pallas_call(
        _head_kernel,
        out_shape=jax.ShapeDtypeStruct((2, N), jnp.float32),
        in_specs=[pl.BlockSpec(memory_space=pltpu.MemorySpace.VMEM)] * 5,
        out_specs=pl.BlockSpec(memory_space=pltpu.MemorySpace.VMEM),
        compiler_params=pltpu.CompilerParams(
            vmem_limit_bytes=32 * 1024 * 1024),
    )(y2, w1m, b1f.reshape(10, 1), w2f, b2f.reshape(2, 1))
    return probs.T


# speedup vs baseline: 19.8181x; 1.0767x over previous
import jax
import jax.numpy as jnp
from jax.experimental import pallas as pl
from jax.experimental.pallas import tpu as pltpu


def _conv_kernel(xq_ref, w1_ref, b1_ref, wa0_ref, wa1_ref, wb1_ref, b2_ref,
                 o_ref, y1_scr):
    n = xq_ref.shape[-1]

    def c1_body(c, carry):
        for ph in range(4):
            for pw in range(4):
                best = None
                for u in range(2):
                    for v in range(2):
                        acc = None
                        for di in range(3):
                            for dj in range(3):
                                sh = 4 * ph + 2 * u + di
                                sw = 4 * pw + 2 * v + dj
                                win = xq_ref[sh % 16, sw % 16,
                                             pl.ds(sh // 16, 8),
                                             pl.ds(sw // 16, 14), :]
                                t = w1_ref[c, 3 * di + dj] * win.astype(
                                    jnp.float32)
                                acc = t if acc is None else acc + t
                        best = acc if best is None else jnp.maximum(best, acc)
                val = jnp.maximum(best + b1_ref[c, 0], 0.0)
                val = jnp.concatenate(
                    [val, jnp.zeros((8, 2, n), jnp.float32)],
                    axis=1).astype(jnp.bfloat16)
                y1_scr[ph, pw, c, :, :, :] = val
        return carry

    jax.lax.fori_loop(0, 16, c1_body, 0)

    m = 7 * 16 * n

    def rhs(u2, v2, di, dj):
        th = 2 * u2 + di
        tw = 2 * v2 + dj
        sl = y1_scr[th % 4, tw % 4, :, pl.ds(th // 4, 7), :, :]
        return sl.reshape(16, m)

    accs = []
    for u2 in range(2):
        for v2 in range(2):
            if v2 == 0:
                r = jnp.concatenate(
                    [rhs(u2, v2, di, dj)
                     for di in range(3) for dj in range(3)], axis=0)
                out = jnp.dot(wa0_ref[...], r,
                              preferred_element_type=jnp.float32)
                acc = out.reshape(32, 7, 16, n)[:, :, 0:13, :]
            else:
                ra = jnp.concatenate(
                    [rhs(u2, v2, di, dj)
                     for di in range(3) for dj in range(2)], axis=0)
                rb = jnp.concatenate(
                    [rhs(u2, v2, di, 2) for di in range(3)], axis=0)
                oa = jnp.dot(wa1_ref[...], ra,
                             preferred_element_type=jnp.float32)
                ob = jnp.dot(wb1_ref[...], rb,
                             preferred_element_type=jnp.float32)
                acc = (oa.reshape(32, 7, 16, n)[:, :, 0:13, :]
                       + ob.reshape(32, 7, 16, n)[:, :, 1:14, :])
            accs.append(acc)
    best = jnp.maximum(jnp.maximum(accs[0], accs[1]),
                       jnp.maximum(accs[2], accs[3]))
    val = jnp.maximum(best + b2_ref[...], 0.0)
    val = jnp.concatenate(
        [val.astype(jnp.bfloat16),
         jnp.zeros((32, 7, 3, n), jnp.bfloat16)], axis=2)
    o_ref[...] = val


def _head_kernel(y2_ref, w1_ref, b1_ref, w2_ref, b2_ref, o_ref):
    n = y2_ref.shape[-1]
    f = y2_ref[:, pl.ds(0, 13), :, :].reshape(32 * 13 * 16, n)
    z = jnp.dot(w1_ref[...], f, preferred_element_type=jnp.float32)
    z = jnp.maximum(z + b1_ref[...], 0.0)
    logits = jnp.dot(w2_ref[...], z,
                     preferred_element_type=jnp.float32) + b2_ref[...]
    mx = jnp.max(logits, axis=0, keepdims=True)
    e = jnp.exp(logits - mx)
    o_ref[...] = e / jnp.sum(e, axis=0, keepdims=True)


def kernel(x, w1c, b1c, w2c, b2c, w1f, b1f, w2f, b2f):
    N = x.shape[0]
    xr = x.reshape(N, 14, 16, 14, 16).transpose(2, 4, 1, 3, 0)
    xq = jnp.pad(xr, ((0, 0), (0, 0), (0, 2), (0, 2), (0, 0)))
    xqs = jnp.concatenate([xq[:, :, 0:9], xq[:, :, 7:16]],
                          axis=2).astype(jnp.bfloat16)

    w2r = w2c.reshape(32, 16, 3, 3).transpose(0, 2, 3, 1)
    wa0 = w2r.reshape(32, 144).astype(jnp.bfloat16)
    wa1 = w2r[:, :, 0:2, :].reshape(32, 96).astype(jnp.bfloat16)
    wb1 = w2r[:, :, 2, :].reshape(32, 48).astype(jnp.bfloat16)

    y2 = pl.pallas_call(
        _conv_kernel,
        out_shape=jax.ShapeDtypeStruct((32, 14, 16, N), jnp.bfloat16),
        grid=(2,),
        in_specs=[
            pl.BlockSpec((16, 16, 9, 16, N), lambda h: (0, 0, h, 0, 0)),
            pl.BlockSpec(memory_space=pltpu.MemorySpace.SMEM),
            pl.BlockSpec(memory_space=pltpu.MemorySpace.SMEM),
            pl.BlockSpec(memory_space=pltpu.MemorySpace.VMEM),
            pl.BlockSpec(memory_space=pltpu.MemorySpace.VMEM),
            pl.BlockSpec(memory_space=pltpu.MemorySpace.VMEM),
            pl.BlockSpec(memory_space=pltpu.MemorySpace.VMEM),
        ],
        out_specs=pl.BlockSpec((32, 7, 16, N), lambda h: (0, h, 0, 0)),
        scratch_shapes=[pltpu.VMEM((4, 4, 16, 8, 16, N), jnp.bfloat16)],
        compiler_params=pltpu.CompilerParams(
            dimension_semantics=("parallel",),
            vmem_limit_bytes=64 * 1024 * 1024),
    )(xqs, w1c, b1c, wa0, wa1, wb1, b2c.reshape(32, 1, 1, 1))

    w1m = jnp.pad(w1f.reshape(10, 32, 13, 13),
                  ((0, 0), (0, 0), (0, 0), (0, 3))
                  ).reshape(10, 32 * 13 * 16).astype(jnp.bfloat16)
    probs = pl.pallas_call(
        _head_kernel,
        out_shape=jax.ShapeDtypeStruct((2, N), jnp.float32),
        in_specs=[pl.BlockSpec(memory_space=pltpu.MemorySpace.VMEM)] * 5,
        out_specs=pl.BlockSpec(memory_space=pltpu.MemorySpace.VMEM),
        compiler_params=pltpu.CompilerParams(
            vmem_limit_bytes=32 * 1024 * 1024),
    )(y2, w1m, b1f.reshape(10, 1), w2f, b2f.reshape(2, 1))
    return probs.T
